# baseline (speedup 1.0000x reference)
import jax
import jax.numpy as jnp
from jax.experimental import pallas as pl

ATOM_FEATS = [7, 5, 4, 4, 2, 2, 4, 3, 8]
DIM = 4
HIDDEN = 64
K = 4
ALPHA = 0.01
NUM_GRAPHS = 512


def _embed(x):
    idx = x[:, :8].astype(jnp.int32)
    outs = [jax.nn.one_hot(idx[:, i], s, dtype=x.dtype) for i, s in enumerate(ATOM_FEATS[:-1])]
    outs.append(x[:, -ATOM_FEATS[-1]:])
    return jnp.concatenate(outs, axis=1)


def _gmm_conv(x, edge_index, edge_attr, g, mu, sigma, root, bias):
    N = x.shape[0]
    M = root.shape[1]
    src, dst = edge_index[0], edge_index[1]
    xg = jnp.matmul(x, g)
    x_j = jnp.take(xg, src, axis=0).reshape(-1, K, M)
    gauss = -0.5 * (edge_attr[:, None, :] - mu[None, :, :]) ** 2
    gauss = gauss / (1e-15 + sigma[None, :, :] ** 2)
    w = jnp.exp(gauss.sum(axis=-1))
    msg = (x_j * w[:, :, None]).sum(axis=1)
    s = jax.ops.segment_sum(msg, dst, num_segments=N)
    cnt = jax.ops.segment_sum(jnp.ones((msg.shape[0],), x.dtype), dst, num_segments=N)
    out = s / jnp.clip(cnt, 1.0, None)[:, None]
    return out + jnp.matmul(x, root) + bias


def _bn(h, eps=1e-5):
    return (h - h.mean(axis=0)) / jnp.sqrt(h.var(axis=0) + eps)


def _layer(x, ei, ea, p, name):
    h = _gmm_conv(x, ei, ea, p['g_' + name], p['mu_' + name], p['sigma_' + name], p['root_' + name], p['bias_' + name])
    return jax.nn.leaky_relu(_bn(h), negative_slope=ALPHA)


def _mlp_kernel(pooled_ref, w1_ref, b1_ref, w2_ref, b2_ref, out_ref):
    pooled = pooled_ref[...]
    h = jnp.dot(pooled, w1_ref[...], preferred_element_type=jnp.float32) + b1_ref[...]
    mean = h.mean(axis=0, keepdims=True)
    var = ((h - mean) ** 2).mean(axis=0, keepdims=True)
    h = (h - mean) / jnp.sqrt(var + 1e-5)
    h = jnp.maximum(h, 0.0)
    out_ref[...] = jnp.dot(h, w2_ref[...], preferred_element_type=jnp.float32) + b2_ref[...]


def kernel(x, edge_index, batch, edge_attr, params):
    h = _embed(x)
    h0 = _layer(h, edge_index, edge_attr, params, 'init')
    x1 = _layer(h0, edge_index, edge_attr, params, 'head')
    x2 = _layer(x1, edge_index, edge_attr, params, 'body')
    x3 = h0 + x2
    x4 = _layer(x3, edge_index, edge_attr, params, 'tail')
    xc = jnp.concatenate((x4, x1, x2, x3), axis=1)
    pooled = jax.ops.segment_max(xc, batch, num_segments=NUM_GRAPHS)
    pooled = jnp.where(jnp.isfinite(pooled), pooled, 0.0)
    out = pl.pallas_call(
        _mlp_kernel,
        out_shape=jax.ShapeDtypeStruct((NUM_GRAPHS, 1), jnp.float32),
    )(pooled, params['w1'], params['b1'][None, :], params['w2'], params['b2'][None, :])
    return out.squeeze(1)


# speedup vs baseline: 2.6737x; 2.6737x over previous
import functools

import jax
import jax.numpy as jnp
from jax import lax
from jax.experimental import pallas as pl
from jax.experimental.pallas import tpu as pltpu
from jax.experimental.pallas import tpu_sc as plsc

ATOM_FEATS = [7, 5, 4, 4, 2, 2, 4, 3, 8]
N = 10000
E = 320000
DIM = 4
HID = 64
K = 4
ALPHA = 0.01
NG = 512

NC, NS = 2, 16
NW = NC * NS
EPW = E // NW
BE = 80
NB = EPW // BE
RPS = 624
RZB = 208
RTAIL = N - NS * RPS

N_PAD = 10240
GPW = NG // NW


def _mesh():
    return plsc.VectorSubcoreMesh(
        core_axis_name="c", subcore_axis_name="s", num_cores=NC, num_subcores=NS)


def _conv_body(xg, src, dst, w, out, src_v, dst_v, w_v, rows_v, msg_v, zbuf, acc_sh, gsem):
    c = lax.axis_index("c")
    s = lax.axis_index("s")
    wid = c * NS + s

    @pl.loop(0, RZB)
    def _zero(i):
        for j in range(HID // 16):
            zbuf[i, pl.ds(16 * j, 16)] = jnp.zeros((16,), jnp.float32)

    for t in range(RPS // RZB):
        pltpu.sync_copy(zbuf, acc_sh.at[pl.ds(s * RPS + t * RZB, RZB)])

    @pl.when(s == NS - 1)
    def _ztail():
        pltpu.sync_copy(zbuf.at[pl.ds(0, RTAIL)], acc_sh.at[pl.ds(NS * RPS, RTAIL)])

    plsc.subcore_barrier()

    @pl.loop(0, NB)
    def _blk(i):
        base = wid * EPW + i * BE
        pltpu.sync_copy(src.at[pl.ds(base, BE)], src_v)
        gcp = pltpu.async_copy(xg.at[src_v], rows_v, gsem)
        for k in range(K):
            pltpu.sync_copy(w.at[pl.ds(k * E + base, BE)], w_v.at[pl.ds(k * BE, BE)])
        pltpu.sync_copy(dst.at[pl.ds(base, BE)], dst_v)
        gcp.wait()

        @pl.loop(0, BE // 16)
        def _grp(gg):
            wk0 = w_v[pl.ds(0 * BE + 16 * gg, 16)]
            wk1 = w_v[pl.ds(1 * BE + 16 * gg, 16)]
            wk2 = w_v[pl.ds(2 * BE + 16 * gg, 16)]
            wk3 = w_v[pl.ds(3 * BE + 16 * gg, 16)]
            for u in range(16):
                e = gg * 16 + u
                w0 = wk0[u]
                w1 = wk1[u]
                w2 = wk2[u]
                w3 = wk3[u]
                for j in range(HID // 16):
                    a = rows_v[e, pl.ds(16 * j, 16)] * w0
                    a = a + rows_v[e, pl.ds(HID + 16 * j, 16)] * w1
                    a = a + rows_v[e, pl.ds(2 * HID + 16 * j, 16)] * w2
                    a = a + rows_v[e, pl.ds(3 * HID + 16 * j, 16)] * w3
                    msg_v[e, pl.ds(16 * j, 16)] = a

        pltpu.sync_copy(msg_v, acc_sh.at[dst_v], add=True)

    plsc.subcore_barrier()
    for t in range(RPS // RZB):
        r0 = s * RPS + t * RZB
        pltpu.sync_copy(acc_sh.at[pl.ds(r0, RZB)], out.at[c, pl.ds(r0, RZB)])

    @pl.when(s == NS - 1)
    def _otail():
        pltpu.sync_copy(acc_sh.at[pl.ds(NS * RPS, RTAIL)], out.at[c, pl.ds(NS * RPS, RTAIL)])


def _conv_sc(xg, src, dst, w):
    f = pl.kernel(
        _conv_body,
        out_type=jax.ShapeDtypeStruct((NC, N, HID), jnp.float32),
        mesh=_mesh(),
        compiler_params=pltpu.CompilerParams(use_tc_tiling_on_sc=False),
        scratch_types=[
            pltpu.VMEM((BE,), jnp.int32),
            pltpu.VMEM((BE,), jnp.int32),
            pltpu.VMEM((BE * K,), jnp.float32),
            pltpu.VMEM((BE, K * HID), jnp.float32),
            pltpu.VMEM((BE, HID), jnp.float32),
            pltpu.VMEM((RZB, HID), jnp.float32),

            pltpu.VMEM_SHARED((N, HID), jnp.float32),
            pltpu.SemaphoreType.DMA,
        ],
    )
    return f(xg, src, dst, w)


def _cnt_body(dst, out, dst_v, ones_v, zbuf, acc_sh):
    c = lax.axis_index("c")
    s = lax.axis_index("s")
    wid = c * NS + s

    @pl.loop(0, RZB)
    def _zero(i):
        zbuf[i, pl.ds(0, 16)] = jnp.zeros((16,), jnp.float32)

    @pl.loop(0, BE)
    def _ones(i):
        ones_v[i, pl.ds(0, 16)] = jnp.ones((16,), jnp.float32)

    for t in range(RPS // RZB):
        pltpu.sync_copy(zbuf, acc_sh.at[pl.ds(s * RPS + t * RZB, RZB)])

    @pl.when(s == NS - 1)
    def _ztail():
        pltpu.sync_copy(zbuf.at[pl.ds(0, RTAIL)], acc_sh.at[pl.ds(NS * RPS, RTAIL)])

    plsc.subcore_barrier()

    @pl.loop(0, NB)
    def _blk(i):
        base = wid * EPW + i * BE
        pltpu.sync_copy(dst.at[pl.ds(base, BE)], dst_v)
        pltpu.sync_copy(ones_v, acc_sh.at[dst_v], add=True)

    plsc.subcore_barrier()
    for t in range(RPS // RZB):
        r0 = s * RPS + t * RZB
        pltpu.sync_copy(acc_sh.at[pl.ds(r0, RZB)], out.at[c, pl.ds(r0, RZB)])

    @pl.when(s == NS - 1)
    def _otail():
        pltpu.sync_copy(acc_sh.at[pl.ds(NS * RPS, RTAIL)], out.at[c, pl.ds(NS * RPS, RTAIL)])


def _cnt_sc(dst):
    f = pl.kernel(
        _cnt_body,
        out_type=jax.ShapeDtypeStruct((NC, N, 16), jnp.float32),
        mesh=_mesh(),
        compiler_params=pltpu.CompilerParams(use_tc_tiling_on_sc=False),
        scratch_types=[
            pltpu.VMEM((BE,), jnp.int32),
            pltpu.VMEM((BE, 16), jnp.float32),
            pltpu.VMEM((RZB, 16), jnp.float32),
            pltpu.VMEM_SHARED((N, 16), jnp.float32),
        ],
    )
    return f(dst)


def _pool_body(xa, xb, xcc, xd, batch, starts, out,
               starts_v, bbuf, fb0, fb1, fb2, fb3, maxbuf):
    c = lax.axis_index("c")
    s = lax.axis_index("s")
    wid = c * NS + s
    g0 = wid * GPW

    pltpu.sync_copy(starts.at[pl.ds(g0, 24)], starts_v.at[pl.ds(0, 24)])
    r0 = starts_v[pl.ds(0, 16)][0]
    r1 = starts_v[pl.ds(GPW, 16)][0]

    @pl.loop(0, GPW)
    def _init(g):
        for j in range(256 // 16):
            maxbuf[g, pl.ds(16 * j, 16)] = jnp.full((16,), -jnp.inf, jnp.float32)

    ra0 = (r0 // 8) * 8
    nblk = (r1 - ra0 + 63) // 64

    @pl.loop(0, nblk)
    def _blk(b):
        row0 = ra0 + b * 64
        pltpu.sync_copy(xa.at[pl.ds(row0, 64)], fb0)
        pltpu.sync_copy(xb.at[pl.ds(row0, 64)], fb1)
        pltpu.sync_copy(xcc.at[pl.ds(row0, 64)], fb2)
        pltpu.sync_copy(xd.at[pl.ds(row0, 64)], fb3)
        pltpu.sync_copy(batch.at[pl.ds(row0, 64)], bbuf.at[pl.ds(0, 64)])

        @pl.loop(0, 64)
        def _row(e):
            row = row0 + e

            @pl.when((row >= r0) & (row < r1))
            def _():
                g = bbuf[pl.ds(e, 16)][0] - g0
                for t, fb in enumerate((fb0, fb1, fb2, fb3)):
                    for j in range(HID // 16):
                        sl = pl.ds(64 * t + 16 * j, 16)
                        maxbuf[g, sl] = jnp.maximum(maxbuf[g, sl], fb[e, pl.ds(16 * j, 16)])

    pltpu.sync_copy(maxbuf, out.at[pl.ds(g0, GPW)])


def _pool_sc(x4, x1, x2, x3, batch_pad, starts_pad):
    f = pl.kernel(
        _pool_body,
        out_type=jax.ShapeDtypeStruct((NG, 4 * HID), jnp.float32),
        mesh=_mesh(),
        compiler_params=pltpu.CompilerParams(use_tc_tiling_on_sc=False),
        scratch_types=[
            pltpu.VMEM((32,), jnp.int32),
            pltpu.VMEM((88,), jnp.int32),
            pltpu.VMEM((64, HID), jnp.float32),
            pltpu.VMEM((64, HID), jnp.float32),
            pltpu.VMEM((64, HID), jnp.float32),
            pltpu.VMEM((64, HID), jnp.float32),
            pltpu.VMEM((GPW, 4 * HID), jnp.float32),
        ],
    )
    return f(x4, x1, x2, x3, batch_pad, starts_pad)


def _wk_body(ea_ref, c_ref, w_ref):
    ea = ea_ref[...]
    bl = ea.shape[1]
    f = jnp.concatenate([ea * ea, ea, jnp.ones((1, bl), jnp.float32)], axis=0)
    w_ref[...] = jnp.exp(-0.5 * jnp.dot(c_ref[...], f, preferred_element_type=jnp.float32, precision=lax.Precision.HIGHEST))


def _w_tc(ea_t, coef):
    grid = 10
    bl = E // grid
    return pl.pallas_call(
        _wk_body,
        grid=(grid,),
        in_specs=[
            pl.BlockSpec((DIM, bl), lambda i: (0, i)),
            pl.BlockSpec((16, 9), lambda i: (0, 0)),
        ],
        out_specs=pl.BlockSpec((16, bl), lambda i: (0, i)),
        out_shape=jax.ShapeDtypeStruct((16, E), jnp.float32),
    )(ea_t, coef)


def _padrows(a):
    return jnp.concatenate(
        [a, jnp.zeros((N_PAD - N, a.shape[1]), jnp.float32)], axis=0)


def _pro_body(x_ref, g_ref, r_ref, xg_ref, rx_ref):
    x = x_ref[...]
    cols = []
    xi = x.astype(jnp.int32)
    for i, sz in enumerate(ATOM_FEATS[:-1]):
        iota = lax.broadcasted_iota(jnp.int32, (N, sz), 1)
        cols.append((xi[:, i:i + 1] == iota).astype(jnp.float32))
    cols.append(x[:, 8:16])
    h = jnp.concatenate(cols, axis=1)
    xg_ref[...] = jnp.dot(h, g_ref[...], preferred_element_type=jnp.float32)
    rx_ref[...] = _padrows(jnp.dot(h, r_ref[...], preferred_element_type=jnp.float32))


def _pro_tc(x, g, r):
    return pl.pallas_call(
        _pro_body,
        out_shape=(
            jax.ShapeDtypeStruct((N, K * HID), jnp.float32),
            jax.ShapeDtypeStruct((N_PAD, HID), jnp.float32),
        ),
    )(x, g, r)


def _combine(acc_ref, cnt_ref, rx_ref, b_ref):
    ssum = acc_ref[0] + acc_ref[1]
    cnt = cnt_ref[0, :, 0:1] + cnt_ref[1, :, 0:1]
    m = ssum / jnp.clip(cnt, 1.0, None) + rx_ref[0:N] + b_ref[...]
    mean = jnp.mean(m, axis=0, keepdims=True)
    var = jnp.mean((m - mean) ** 2, axis=0, keepdims=True)
    h = (m - mean) / jnp.sqrt(var + 1e-5)
    return jnp.where(h >= 0, h, ALPHA * h)


def _ca_body(acc_ref, cnt_ref, rx_ref, b_ref, h_ref):
    h_ref[...] = _padrows(_combine(acc_ref, cnt_ref, rx_ref, b_ref))


def _ca_tc(acc, cnt2, rx, bias):
    return pl.pallas_call(
        _ca_body,
        out_shape=jax.ShapeDtypeStruct((N_PAD, HID), jnp.float32),
    )(acc, cnt2, rx, bias)


def _cb_body(acc_ref, cnt_ref, rx_ref, b_ref, h0_ref, x2_ref, x3_ref):
    x2 = _padrows(_combine(acc_ref, cnt_ref, rx_ref, b_ref))
    x2_ref[...] = x2
    x3_ref[...] = h0_ref[...] + x2


def _cb_tc(acc, cnt2, rx, bias, h0):
    return pl.pallas_call(
        _cb_body,
        out_shape=(
            jax.ShapeDtypeStruct((N_PAD, HID), jnp.float32),
            jax.ShapeDtypeStruct((N_PAD, HID), jnp.float32),
        ),
    )(acc, cnt2, rx, bias, h0)


def _mm_body(h_ref, g_ref, r_ref, xg_ref, rx_ref):
    h = h_ref[...]
    xg_ref[...] = jnp.dot(h, g_ref[...], preferred_element_type=jnp.float32)
    rx_ref[...] = jnp.dot(h, r_ref[...], preferred_element_type=jnp.float32)


def _mm_tc(h, g_next, root_next):
    return pl.pallas_call(
        _mm_body,
        out_shape=(
            jax.ShapeDtypeStruct((N_PAD, K * HID), jnp.float32),
            jax.ShapeDtypeStruct((N_PAD, HID), jnp.float32),
        ),
    )(h, g_next, root_next)


def _mlp_body(p_ref, w1_ref, b1_ref, w2_ref, b2_ref, out_ref):
    p = p_ref[...]
    p = jnp.where(jnp.isfinite(p), p, 0.0)
    h = jnp.dot(p, w1_ref[...], preferred_element_type=jnp.float32) + b1_ref[...]
    mean = jnp.mean(h, axis=0, keepdims=True)
    var = jnp.mean((h - mean) ** 2, axis=0, keepdims=True)
    h = (h - mean) / jnp.sqrt(var + 1e-5)
    h = jnp.maximum(h, 0.0)
    out_ref[...] = jnp.dot(h, w2_ref[...], preferred_element_type=jnp.float32) + b2_ref[...]


def _mlp_tc(pooled, w1, b1, w2, b2):
    return pl.pallas_call(
        _mlp_body,
        out_shape=jax.ShapeDtypeStruct((NG, 1), jnp.float32),
    )(pooled, w1, b1, w2, b2)


def kernel(x, edge_index, batch, edge_attr, params):
    p = params
    src = edge_index[0]
    dst = edge_index[1]

    mu_all = jnp.concatenate([p['mu_' + n] for n in ('init', 'head', 'body', 'tail')], 0)
    sig_all = jnp.concatenate([p['sigma_' + n] for n in ('init', 'head', 'body', 'tail')], 0)
    inv_all = 1.0 / (1e-15 + sig_all ** 2)
    coef = jnp.concatenate(
        [inv_all, -2.0 * inv_all * mu_all,
         jnp.sum(inv_all * mu_all ** 2, axis=1, keepdims=True)], axis=1)
    w_t = _w_tc(edge_attr.T, coef)

    cnt2 = _cnt_sc(dst)

    xg0, rx0 = _pro_tc(x, p['g_init'], p['root_init'])

    acc0 = _conv_sc(xg0, src, dst, w_t[0:4].reshape(-1))
    h0 = _ca_tc(acc0, cnt2, rx0, p['bias_init'][None, :])
    xg1, rx1 = _mm_tc(h0, p['g_head'], p['root_head'])
    acc1 = _conv_sc(xg1, src, dst, w_t[4:8].reshape(-1))
    x1 = _ca_tc(acc1, cnt2, rx1, p['bias_head'][None, :])
    xg2, rx2 = _mm_tc(x1, p['g_body'], p['root_body'])
    acc2 = _conv_sc(xg2, src, dst, w_t[8:12].reshape(-1))
    x2, x3 = _cb_tc(acc2, cnt2, rx2, p['bias_body'][None, :], h0)
    xg3, rx3 = _mm_tc(x3, p['g_tail'], p['root_tail'])
    acc3 = _conv_sc(xg3, src, dst, w_t[12:16].reshape(-1))
    x4 = _ca_tc(acc3, cnt2, rx3, p['bias_tail'][None, :])

    starts = jnp.searchsorted(batch, jnp.arange(NG + 1, dtype=jnp.int32)).astype(jnp.int32)
    starts_pad = jnp.concatenate([starts, jnp.full((15,), N, jnp.int32)])
    batch_pad = jnp.concatenate([batch, jnp.full((N_PAD - N,), NG, jnp.int32)])
    pooled = _pool_sc(x4, x1, x2, x3, batch_pad, starts_pad)

    out = _mlp_tc(pooled, p['w1'], p['b1'][None, :], p['w2'], p['b2'][None, :])
    return out[:, 0]


# speedup vs baseline: 4.1220x; 1.5417x over previous
import functools

import jax
import jax.numpy as jnp
from jax import lax
from jax.experimental import pallas as pl
from jax.experimental.pallas import tpu as pltpu
from jax.experimental.pallas import tpu_sc as plsc

ATOM_FEATS = [7, 5, 4, 4, 2, 2, 4, 3, 8]
N = 10000
E = 320000
DIM = 4
HID = 64
K = 4
ALPHA = 0.01
NG = 512

NC, NS = 2, 16
NW = NC * NS
EPW = E // NW
BE = 80
NB = EPW // BE
RPS = 624
RZB = 208
RTAIL = N - NS * RPS

N_PAD = 10240
GPW = NG // NW


def _mesh():
    return plsc.VectorSubcoreMesh(
        core_axis_name="c", subcore_axis_name="s", num_cores=NC, num_subcores=NS)


ZCH = 104


def _conv_body(xg, src, dst, w, out, src_all, dst_all, wbuf_a, wbuf_b, dst_v,
               rows_a, rows_b, msg_v, zbuf, acc_sh, sem_a, sem_b):
    c = lax.axis_index("c")
    s = lax.axis_index("s")
    wid = c * NS + s
    ebase = wid * EPW

    pltpu.sync_copy(src.at[pl.ds(ebase, EPW)], src_all)
    pltpu.sync_copy(dst.at[pl.ds(ebase, EPW)], dst_all)

    @pl.loop(0, ZCH)
    def _zero(i):
        for j in range(HID // 16):
            zbuf[i, pl.ds(16 * j, 16)] = jnp.zeros((16,), jnp.float32)

    for t in range(RPS // ZCH):
        pltpu.sync_copy(zbuf, acc_sh.at[pl.ds(s * RPS + t * ZCH, ZCH)])

    @pl.when(s == NS - 1)
    def _ztail():
        pltpu.sync_copy(zbuf.at[pl.ds(0, RTAIL)], acc_sh.at[pl.ds(NS * RPS, RTAIL)])

    plsc.subcore_barrier()

    def issue(b, rows, wbuf, sem):
        pltpu.async_copy(xg.at[src_all.at[pl.ds(b * BE, BE)]], rows, sem)
        pltpu.async_copy(w.at[pl.ds((wid * NB + b) * K * BE, K * BE)], wbuf, sem)

    def wait(b, rows, wbuf, sem):
        pltpu.make_async_copy(xg.at[src_all.at[pl.ds(b * BE, BE)]], rows, sem).wait()
        pltpu.make_async_copy(w.at[pl.ds(0, K * BE)], wbuf, sem).wait()

    def process(b, rows, wbuf):
        for j in range(BE // 16):
            dst_v[pl.ds(16 * j, 16)] = dst_all[pl.ds(b * BE + 16 * j, 16)]

        @pl.loop(0, BE // 16)
        def _grp(gg):
            wk0 = wbuf[pl.ds(0 * BE + 16 * gg, 16)]
            wk1 = wbuf[pl.ds(1 * BE + 16 * gg, 16)]
            wk2 = wbuf[pl.ds(2 * BE + 16 * gg, 16)]
            wk3 = wbuf[pl.ds(3 * BE + 16 * gg, 16)]
            for u in range(16):
                e = gg * 16 + u
                w0 = wk0[u]
                w1 = wk1[u]
                w2 = wk2[u]
                w3 = wk3[u]
                for j in range(HID // 16):
                    a = rows[e, pl.ds(16 * j, 16)] * w0
                    a = a + rows[e, pl.ds(HID + 16 * j, 16)] * w1
                    a = a + rows[e, pl.ds(2 * HID + 16 * j, 16)] * w2
                    a = a + rows[e, pl.ds(3 * HID + 16 * j, 16)] * w3
                    msg_v[e, pl.ds(16 * j, 16)] = a

        pltpu.sync_copy(msg_v, acc_sh.at[dst_v], add=True)

    issue(0, rows_a, wbuf_a, sem_a)

    @pl.loop(0, NB // 2)
    def _pair(ii):
        b0 = 2 * ii
        issue(b0 + 1, rows_b, wbuf_b, sem_b)
        wait(b0, rows_a, wbuf_a, sem_a)
        process(b0, rows_a, wbuf_a)
        issue(b0 + 2, rows_a, wbuf_a, sem_a)
        wait(b0 + 1, rows_b, wbuf_b, sem_b)
        process(b0 + 1, rows_b, wbuf_b)

    wait(NB - 1, rows_a, wbuf_a, sem_a)
    process(NB - 1, rows_a, wbuf_a)

    plsc.subcore_barrier()
    for t in range(RPS // RZB):
        r0 = s * RPS + t * RZB
        pltpu.sync_copy(acc_sh.at[pl.ds(r0, RZB)], out.at[c, pl.ds(r0, RZB)])

    @pl.when(s == NS - 1)
    def _otail():
        pltpu.sync_copy(acc_sh.at[pl.ds(NS * RPS, RTAIL)], out.at[c, pl.ds(NS * RPS, RTAIL)])


def _conv_sc(xg, src, dst, w):
    f = pl.kernel(
        _conv_body,
        out_type=jax.ShapeDtypeStruct((NC, N, HID), jnp.float32),
        mesh=_mesh(),
        compiler_params=pltpu.CompilerParams(use_tc_tiling_on_sc=False),
        scratch_types=[
            pltpu.VMEM((EPW,), jnp.int32),
            pltpu.VMEM((EPW,), jnp.int32),
            pltpu.VMEM((K * BE,), jnp.float32),
            pltpu.VMEM((K * BE,), jnp.float32),
            pltpu.VMEM((BE,), jnp.int32),
            pltpu.VMEM((BE, K * HID), jnp.float32),
            pltpu.VMEM((BE, K * HID), jnp.float32),
            pltpu.VMEM((BE, HID), jnp.float32),
            pltpu.VMEM((ZCH, HID), jnp.float32),
            pltpu.VMEM_SHARED((N, HID), jnp.float32),
            pltpu.SemaphoreType.DMA,
            pltpu.SemaphoreType.DMA,
        ],
    )
    return f(xg, src, dst, w)


def _cnt_body(dst, out, dst_v, ones_v, zbuf, acc_sh):
    c = lax.axis_index("c")
    s = lax.axis_index("s")
    wid = c * NS + s

    @pl.loop(0, RZB)
    def _zero(i):
        zbuf[i, pl.ds(0, 16)] = jnp.zeros((16,), jnp.float32)

    @pl.loop(0, BE)
    def _ones(i):
        ones_v[i, pl.ds(0, 16)] = jnp.ones((16,), jnp.float32)

    for t in range(RPS // RZB):
        pltpu.sync_copy(zbuf, acc_sh.at[pl.ds(s * RPS + t * RZB, RZB)])

    @pl.when(s == NS - 1)
    def _ztail():
        pltpu.sync_copy(zbuf.at[pl.ds(0, RTAIL)], acc_sh.at[pl.ds(NS * RPS, RTAIL)])

    plsc.subcore_barrier()

    @pl.loop(0, NB)
    def _blk(i):
        base = wid * EPW + i * BE
        pltpu.sync_copy(dst.at[pl.ds(base, BE)], dst_v)
        pltpu.sync_copy(ones_v, acc_sh.at[dst_v], add=True)

    plsc.subcore_barrier()
    for t in range(RPS // RZB):
        r0 = s * RPS + t * RZB
        pltpu.sync_copy(acc_sh.at[pl.ds(r0, RZB)], out.at[c, pl.ds(r0, RZB)])

    @pl.when(s == NS - 1)
    def _otail():
        pltpu.sync_copy(acc_sh.at[pl.ds(NS * RPS, RTAIL)], out.at[c, pl.ds(NS * RPS, RTAIL)])


def _cnt_sc(dst):
    f = pl.kernel(
        _cnt_body,
        out_type=jax.ShapeDtypeStruct((NC, N, 16), jnp.float32),
        mesh=_mesh(),
        compiler_params=pltpu.CompilerParams(use_tc_tiling_on_sc=False),
        scratch_types=[
            pltpu.VMEM((BE,), jnp.int32),
            pltpu.VMEM((BE, 16), jnp.float32),
            pltpu.VMEM((RZB, 16), jnp.float32),
            pltpu.VMEM_SHARED((N, 16), jnp.float32),
        ],
    )
    return f(dst)


def _pool_body(xa, xb, xcc, xd, batch, starts, out,
               starts_v, bbuf, fb0, fb1, fb2, fb3, maxbuf):
    c = lax.axis_index("c")
    s = lax.axis_index("s")
    wid = c * NS + s
    g0 = wid * GPW

    pltpu.sync_copy(starts.at[pl.ds(g0, 24)], starts_v.at[pl.ds(0, 24)])
    r0 = starts_v[pl.ds(0, 16)][0]
    r1 = starts_v[pl.ds(GPW, 16)][0]

    @pl.loop(0, GPW)
    def _init(g):
        for j in range(256 // 16):
            maxbuf[g, pl.ds(16 * j, 16)] = jnp.full((16,), -jnp.inf, jnp.float32)

    ra0 = (r0 // 8) * 8
    nblk = (r1 - ra0 + 63) // 64

    @pl.loop(0, nblk)
    def _blk(b):
        row0 = ra0 + b * 64
        pltpu.sync_copy(xa.at[pl.ds(row0, 64)], fb0)
        pltpu.sync_copy(xb.at[pl.ds(row0, 64)], fb1)
        pltpu.sync_copy(xcc.at[pl.ds(row0, 64)], fb2)
        pltpu.sync_copy(xd.at[pl.ds(row0, 64)], fb3)
        pltpu.sync_copy(batch.at[pl.ds(row0, 64)], bbuf.at[pl.ds(0, 64)])

        @pl.loop(0, 64)
        def _row(e):
            row = row0 + e

            @pl.when((row >= r0) & (row < r1))
            def _():
                g = bbuf[pl.ds(e, 16)][0] - g0
                for t, fb in enumerate((fb0, fb1, fb2, fb3)):
                    for j in range(HID // 16):
                        sl = pl.ds(64 * t + 16 * j, 16)
                        maxbuf[g, sl] = jnp.maximum(maxbuf[g, sl], fb[e, pl.ds(16 * j, 16)])

    pltpu.sync_copy(maxbuf, out.at[pl.ds(g0, GPW)])


def _pool_sc(x4, x1, x2, x3, batch_pad, starts_pad):
    f = pl.kernel(
        _pool_body,
        out_type=jax.ShapeDtypeStruct((NG, 4 * HID), jnp.float32),
        mesh=_mesh(),
        compiler_params=pltpu.CompilerParams(use_tc_tiling_on_sc=False),
        scratch_types=[
            pltpu.VMEM((32,), jnp.int32),
            pltpu.VMEM((88,), jnp.int32),
            pltpu.VMEM((64, HID), jnp.float32),
            pltpu.VMEM((64, HID), jnp.float32),
            pltpu.VMEM((64, HID), jnp.float32),
            pltpu.VMEM((64, HID), jnp.float32),
            pltpu.VMEM((GPW, 4 * HID), jnp.float32),
        ],
    )
    return f(x4, x1, x2, x3, batch_pad, starts_pad)


def _wk_body(ea_ref, c_ref, w_ref):
    ea = ea_ref[...]
    bl = ea.shape[1]
    f = jnp.concatenate([ea * ea, ea, jnp.ones((1, bl), jnp.float32)], axis=0)
    w_ref[...] = jnp.exp(-0.5 * jnp.dot(c_ref[...], f, preferred_element_type=jnp.float32, precision=lax.Precision.HIGHEST))


def _w_tc(ea_t, coef):
    grid = 10
    bl = E // grid
    return pl.pallas_call(
        _wk_body,
        grid=(grid,),
        in_specs=[
            pl.BlockSpec((DIM, bl), lambda i: (0, i)),
            pl.BlockSpec((16, 9), lambda i: (0, 0)),
        ],
        out_specs=pl.BlockSpec((16, bl), lambda i: (0, i)),
        out_shape=jax.ShapeDtypeStruct((16, E), jnp.float32),
    )(ea_t, coef)


def _padrows(a):
    return jnp.concatenate(
        [a, jnp.zeros((N_PAD - N, a.shape[1]), jnp.float32)], axis=0)


def _pro_body(x_ref, g_ref, r_ref, xg_ref, rx_ref):
    x = x_ref[...]
    cols = []
    xi = x.astype(jnp.int32)
    for i, sz in enumerate(ATOM_FEATS[:-1]):
        iota = lax.broadcasted_iota(jnp.int32, (N, sz), 1)
        cols.append((xi[:, i:i + 1] == iota).astype(jnp.float32))
    cols.append(x[:, 8:16])
    h = jnp.concatenate(cols, axis=1)
    xg_ref[...] = jnp.dot(h, g_ref[...], preferred_element_type=jnp.float32)
    rx_ref[...] = _padrows(jnp.dot(h, r_ref[...], preferred_element_type=jnp.float32))


def _pro_tc(x, g, r):
    return pl.pallas_call(
        _pro_body,
        out_shape=(
            jax.ShapeDtypeStruct((N, K * HID), jnp.float32),
            jax.ShapeDtypeStruct((N_PAD, HID), jnp.float32),
        ),
    )(x, g, r)


def _combine(acc_ref, cnt_ref, rx_ref, b_ref):
    ssum = acc_ref[0] + acc_ref[1]
    cnt = cnt_ref[0, :, 0:1] + cnt_ref[1, :, 0:1]
    m = ssum / jnp.clip(cnt, 1.0, None) + rx_ref[0:N] + b_ref[...]
    mean = jnp.mean(m, axis=0, keepdims=True)
    var = jnp.mean((m - mean) ** 2, axis=0, keepdims=True)
    h = (m - mean) / jnp.sqrt(var + 1e-5)
    return jnp.where(h >= 0, h, ALPHA * h)


def _ca_body(acc_ref, cnt_ref, rx_ref, b_ref, h_ref):
    h_ref[...] = _padrows(_combine(acc_ref, cnt_ref, rx_ref, b_ref))


def _ca_tc(acc, cnt2, rx, bias):
    return pl.pallas_call(
        _ca_body,
        out_shape=jax.ShapeDtypeStruct((N_PAD, HID), jnp.float32),
    )(acc, cnt2, rx, bias)


def _cb_body(acc_ref, cnt_ref, rx_ref, b_ref, h0_ref, x2_ref, x3_ref):
    x2 = _padrows(_combine(acc_ref, cnt_ref, rx_ref, b_ref))
    x2_ref[...] = x2
    x3_ref[...] = h0_ref[...] + x2


def _cb_tc(acc, cnt2, rx, bias, h0):
    return pl.pallas_call(
        _cb_body,
        out_shape=(
            jax.ShapeDtypeStruct((N_PAD, HID), jnp.float32),
            jax.ShapeDtypeStruct((N_PAD, HID), jnp.float32),
        ),
    )(acc, cnt2, rx, bias, h0)


def _mm_body(h_ref, g_ref, r_ref, xg_ref, rx_ref):
    h = h_ref[...]
    xg_ref[...] = jnp.dot(h, g_ref[...], preferred_element_type=jnp.float32)
    rx_ref[...] = jnp.dot(h, r_ref[...], preferred_element_type=jnp.float32)


def _mm_tc(h, g_next, root_next):
    return pl.pallas_call(
        _mm_body,
        out_shape=(
            jax.ShapeDtypeStruct((N_PAD, K * HID), jnp.float32),
            jax.ShapeDtypeStruct((N_PAD, HID), jnp.float32),
        ),
    )(h, g_next, root_next)


def _mlp_body(p_ref, w1_ref, b1_ref, w2_ref, b2_ref, out_ref):
    p = p_ref[...]
    p = jnp.where(jnp.isfinite(p), p, 0.0)
    h = jnp.dot(p, w1_ref[...], preferred_element_type=jnp.float32) + b1_ref[...]
    mean = jnp.mean(h, axis=0, keepdims=True)
    var = jnp.mean((h - mean) ** 2, axis=0, keepdims=True)
    h = (h - mean) / jnp.sqrt(var + 1e-5)
    h = jnp.maximum(h, 0.0)
    out_ref[...] = jnp.dot(h, w2_ref[...], preferred_element_type=jnp.float32) + b2_ref[...]


def _mlp_tc(pooled, w1, b1, w2, b2):
    return pl.pallas_call(
        _mlp_body,
        out_shape=jax.ShapeDtypeStruct((NG, 1), jnp.float32),
    )(pooled, w1, b1, w2, b2)


def kernel(x, edge_index, batch, edge_attr, params):
    p = params
    src = edge_index[0]
    dst = edge_index[1]

    mu_all = jnp.concatenate([p['mu_' + n] for n in ('init', 'head', 'body', 'tail')], 0)
    sig_all = jnp.concatenate([p['sigma_' + n] for n in ('init', 'head', 'body', 'tail')], 0)
    inv_all = 1.0 / (1e-15 + sig_all ** 2)
    coef = jnp.concatenate(
        [inv_all, -2.0 * inv_all * mu_all,
         jnp.sum(inv_all * mu_all ** 2, axis=1, keepdims=True)], axis=1)
    w_t = _w_tc(edge_attr.T, coef)
    w_blk = w_t.reshape(4, K, NW, NB, BE).transpose(0, 2, 3, 1, 4).reshape(4, -1)

    cnt2 = _cnt_sc(dst)

    xg0, rx0 = _pro_tc(x, p['g_init'], p['root_init'])

    acc0 = _conv_sc(xg0, src, dst, w_blk[0])
    h0 = _ca_tc(acc0, cnt2, rx0, p['bias_init'][None, :])
    xg1, rx1 = _mm_tc(h0, p['g_head'], p['root_head'])
    acc1 = _conv_sc(xg1, src, dst, w_blk[1])
    x1 = _ca_tc(acc1, cnt2, rx1, p['bias_head'][None, :])
    xg2, rx2 = _mm_tc(x1, p['g_body'], p['root_body'])
    acc2 = _conv_sc(xg2, src, dst, w_blk[2])
    x2, x3 = _cb_tc(acc2, cnt2, rx2, p['bias_body'][None, :], h0)
    xg3, rx3 = _mm_tc(x3, p['g_tail'], p['root_tail'])
    acc3 = _conv_sc(xg3, src, dst, w_blk[3])
    x4 = _ca_tc(acc3, cnt2, rx3, p['bias_tail'][None, :])

    starts = jnp.searchsorted(batch, jnp.arange(NG + 1, dtype=jnp.int32)).astype(jnp.int32)
    starts_pad = jnp.concatenate([starts, jnp.full((15,), N, jnp.int32)])
    batch_pad = jnp.concatenate([batch, jnp.full((N_PAD - N,), NG, jnp.int32)])
    pooled = _pool_sc(x4, x1, x2, x3, batch_pad, starts_pad)

    out = _mlp_tc(pooled, p['w1'], p['b1'][None, :], p['w2'], p['b2'][None, :])
    return out[:, 0]
